# baseline (device time: 43225 ns/iter reference)
import jax
import jax.numpy as jnp
from jax import lax
from jax.experimental import pallas as pl
from jax.experimental.pallas import tpu as pltpu

N_ROWS = 1024
N_I32 = 256


def _body(x_ref, d_ref, p_ref, out_ref, send_sem, recv_sem):
    my_x = lax.axis_index("x")
    my_y = lax.axis_index("y")
    peer_y = 1 - my_y

    barrier = pltpu.get_barrier_semaphore()
    pl.semaphore_signal(
        barrier, inc=1,
        device_id=(my_x, peer_y),
        device_id_type=pl.DeviceIdType.MESH,
    )
    pl.semaphore_wait(barrier, 1)

    def issue(i, _):
        d = d_ref[i]
        p = p_ref[i]
        rdma = pltpu.make_async_remote_copy(
            src_ref=x_ref.at[pl.ds(i, 1)],
            dst_ref=out_ref.at[pl.ds(p, 1)],
            send_sem=send_sem,
            recv_sem=recv_sem,
            device_id=(my_x, d),
            device_id_type=pl.DeviceIdType.MESH,
        )
        rdma.start()
        return 0

    lax.fori_loop(0, N_ROWS, issue, 0)

    def _dummy():
        return pltpu.make_async_remote_copy(
            src_ref=x_ref.at[pl.ds(0, 1)],
            dst_ref=out_ref.at[pl.ds(0, 1)],
            send_sem=send_sem,
            recv_sem=recv_sem,
            device_id=(my_x, my_y),
            device_id_type=pl.DeviceIdType.MESH,
        )

    def wait_recv(i, _):
        _dummy().wait_recv()
        return 0

    def wait_send(i, _):
        _dummy().wait_send()
        return 0

    lax.fori_loop(0, N_ROWS, wait_recv, 0)
    lax.fori_loop(0, N_ROWS, wait_send, 0)


def kernel(x, dest):
    my_y = lax.axis_index("y")

    dest = dest.astype(jnp.int32)
    ones = (dest == 1).astype(jnp.int32)
    c1 = jnp.cumsum(ones)
    idx = jnp.arange(N_ROWS, dtype=jnp.int32)
    rank = jnp.where(ones == 1, c1 - 1, idx - c1)
    tot1 = c1[-1]
    cnt = jnp.where(ones == 1, tot1, N_ROWS - tot1)
    pos = jnp.where(my_y == 0, rank, N_ROWS - cnt + rank).astype(jnp.int32)

    xb = x.astype(jnp.bfloat16)
    xi = jax.lax.bitcast_convert_type(
        xb.reshape(N_ROWS, N_I32, 2), jnp.int32
    )

    out = pl.pallas_call(
        _body,
        out_shape=jax.ShapeDtypeStruct((N_ROWS, N_I32), jnp.int32),
        in_specs=[
            pl.BlockSpec(memory_space=pltpu.VMEM),
            pl.BlockSpec(memory_space=pltpu.SMEM),
            pl.BlockSpec(memory_space=pltpu.SMEM),
        ],
        out_specs=pl.BlockSpec(memory_space=pltpu.VMEM),
        scratch_shapes=[
            pltpu.SemaphoreType.DMA,
            pltpu.SemaphoreType.DMA,
        ],
        compiler_params=pltpu.CompilerParams(collective_id=0),
    )(xi, dest, pos)

    return jax.lax.bitcast_convert_type(out, jnp.bfloat16).reshape(
        N_ROWS, 2 * N_I32
    )


# device time: 34270 ns/iter; 1.2613x vs baseline; 1.2613x over previous
import jax
import jax.numpy as jnp
from jax import lax
from jax.experimental import pallas as pl
from jax.experimental.pallas import tpu as pltpu

N_ROWS = 1024
N_I32 = 256


def _body(x_ref, d_ref, p_ref, out_ref, send_sem, recv_sem):
    my_x = lax.axis_index("x")
    my_y = lax.axis_index("y")
    peer_y = 1 - my_y

    barrier = pltpu.get_barrier_semaphore()
    pl.semaphore_signal(
        barrier, inc=1,
        device_id=(my_x, peer_y),
        device_id_type=pl.DeviceIdType.MESH,
    )
    pl.semaphore_wait(barrier, 1)

    def issue(i, _):
        d = d_ref[i]
        p = p_ref[i]
        rdma = pltpu.make_async_remote_copy(
            src_ref=x_ref.at[pl.ds(i, 1)],
            dst_ref=out_ref.at[pl.ds(p, 1)],
            send_sem=send_sem,
            recv_sem=recv_sem,
            device_id=(my_x, d),
            device_id_type=pl.DeviceIdType.MESH,
        )
        rdma.start()
        return 0

    lax.fori_loop(0, N_ROWS, issue, 0, unroll=8)

    full = pltpu.make_async_remote_copy(
        src_ref=x_ref,
        dst_ref=out_ref,
        send_sem=send_sem,
        recv_sem=recv_sem,
        device_id=(my_x, my_y),
        device_id_type=pl.DeviceIdType.MESH,
    )
    full.wait_recv()
    full.wait_send()


def kernel(x, dest):
    my_y = lax.axis_index("y")

    dest = dest.astype(jnp.int32)
    ones = (dest == 1).astype(jnp.int32)
    c1 = jnp.cumsum(ones)
    idx = jnp.arange(N_ROWS, dtype=jnp.int32)
    rank = jnp.where(ones == 1, c1 - 1, idx - c1)
    tot1 = c1[-1]
    cnt = jnp.where(ones == 1, tot1, N_ROWS - tot1)
    pos = jnp.where(my_y == 0, rank, N_ROWS - cnt + rank).astype(jnp.int32)

    xb = x.astype(jnp.bfloat16)
    xi = jax.lax.bitcast_convert_type(
        xb.reshape(N_ROWS, N_I32, 2), jnp.int32
    )

    out = pl.pallas_call(
        _body,
        out_shape=jax.ShapeDtypeStruct((N_ROWS, N_I32), jnp.int32),
        in_specs=[
            pl.BlockSpec(memory_space=pltpu.VMEM),
            pl.BlockSpec(memory_space=pltpu.SMEM),
            pl.BlockSpec(memory_space=pltpu.SMEM),
        ],
        out_specs=pl.BlockSpec(memory_space=pltpu.VMEM),
        scratch_shapes=[
            pltpu.SemaphoreType.DMA,
            pltpu.SemaphoreType.DMA,
        ],
        compiler_params=pltpu.CompilerParams(collective_id=0),
    )(xi, dest, pos)

    return jax.lax.bitcast_convert_type(out, jnp.bfloat16).reshape(
        N_ROWS, 2 * N_I32
    )


# device time: 16865 ns/iter; 2.5630x vs baseline; 2.0320x over previous
import jax
import jax.numpy as jnp
from jax import lax
from jax.experimental import pallas as pl
from jax.experimental.pallas import tpu as pltpu

N_ROWS = 1024
N_COLS = 512
CHUNK = 64
N_CHUNKS = N_ROWS // CHUNK


def _body(x_ref, meta_ref, kpos_ref, spos_ref, out_ref,
          s_ref, r_ref, send_sem, recv_sem):
    my_x = lax.axis_index("x")
    my_y = lax.axis_index("y")
    peer = (my_x, 1 - my_y)

    r_ref[...] = jnp.zeros((N_ROWS, N_COLS), jnp.bfloat16)

    barrier = pltpu.get_barrier_semaphore()
    pl.semaphore_signal(
        barrier, inc=1, device_id=peer,
        device_id_type=pl.DeviceIdType.MESH,
    )
    pl.semaphore_wait(barrier, 1)

    xb = x_ref[...].astype(jnp.bfloat16)
    iota = lax.broadcasted_iota(jnp.int32, (N_ROWS, N_ROWS), 0)

    p_send = (spos_ref[...] == iota).astype(jnp.bfloat16)
    s_ref[...] = jnp.dot(
        p_send, xb, preferred_element_type=jnp.float32
    ).astype(jnp.bfloat16)

    lo_s = meta_ref[0]
    hi_s = meta_ref[1]
    lo_r = meta_ref[2]
    hi_r = meta_ref[3]

    def chunk_rdma(j):
        return pltpu.make_async_remote_copy(
            src_ref=s_ref.at[pl.ds(j * CHUNK, CHUNK)],
            dst_ref=r_ref.at[pl.ds(j * CHUNK, CHUNK)],
            send_sem=send_sem,
            recv_sem=recv_sem,
            device_id=peer,
            device_id_type=pl.DeviceIdType.MESH,
        )

    for j in range(N_CHUNKS):
        @pl.when(jnp.logical_and(j * CHUNK < hi_s, (j + 1) * CHUNK > lo_s))
        def _(j=j):
            chunk_rdma(j).start()

    p_keep = (kpos_ref[...] == iota).astype(jnp.bfloat16)
    keep = jnp.dot(
        p_keep, xb, preferred_element_type=jnp.float32
    ).astype(jnp.bfloat16)

    for j in range(N_CHUNKS):
        @pl.when(jnp.logical_and(j * CHUNK < hi_r, (j + 1) * CHUNK > lo_r))
        def _():
            chunk_rdma(0).wait_recv()

    out_ref[...] = keep + r_ref[...]

    for j in range(N_CHUNKS):
        @pl.when(jnp.logical_and(j * CHUNK < hi_s, (j + 1) * CHUNK > lo_s))
        def _():
            chunk_rdma(0).wait_send()


def kernel(x, dest):
    my_y = lax.axis_index("y")

    dest = dest.astype(jnp.int32)
    ones = (dest == 1).astype(jnp.int32)
    c1 = jnp.cumsum(ones)
    idx = jnp.arange(N_ROWS, dtype=jnp.int32)
    rank = jnp.where(ones == 1, c1 - 1, idx - c1)
    tot1 = c1[-1]
    cnt = jnp.where(ones == 1, tot1, N_ROWS - tot1)
    pos = jnp.where(my_y == 0, rank, N_ROWS - cnt + rank).astype(jnp.int32)

    keeppos = jnp.where(dest == my_y, pos, -1).reshape(1, N_ROWS)
    sendpos = jnp.where(dest != my_y, pos, -1).reshape(1, N_ROWS)

    c_send = jnp.where(my_y == 0, tot1, N_ROWS - tot1)
    k_recv = c_send

    lo_send = jnp.where(my_y == 0, 0, N_ROWS - c_send)
    hi_send = jnp.where(my_y == 0, c_send, N_ROWS)
    lo_recv = jnp.where(my_y == 0, N_ROWS - k_recv, 0)
    hi_recv = jnp.where(my_y == 0, N_ROWS, k_recv)
    meta = jnp.stack([lo_send, hi_send, lo_recv, hi_recv]).astype(jnp.int32)

    return pl.pallas_call(
        _body,
        out_shape=jax.ShapeDtypeStruct((N_ROWS, N_COLS), jnp.bfloat16),
        in_specs=[
            pl.BlockSpec(memory_space=pltpu.VMEM),
            pl.BlockSpec(memory_space=pltpu.SMEM),
            pl.BlockSpec(memory_space=pltpu.VMEM),
            pl.BlockSpec(memory_space=pltpu.VMEM),
        ],
        out_specs=pl.BlockSpec(memory_space=pltpu.VMEM),
        scratch_shapes=[
            pltpu.VMEM((N_ROWS, N_COLS), jnp.bfloat16),
            pltpu.VMEM((N_ROWS, N_COLS), jnp.bfloat16),
            pltpu.SemaphoreType.DMA,
            pltpu.SemaphoreType.DMA,
        ],
        compiler_params=pltpu.CompilerParams(collective_id=0),
    )(x, meta, keeppos, sendpos)


# device time: 15353 ns/iter; 2.8154x vs baseline; 1.0985x over previous
import jax
import jax.numpy as jnp
from jax import lax
from jax.experimental import pallas as pl
from jax.experimental.pallas import tpu as pltpu

N_ROWS = 1024
N_COLS = 512
CHUNK = 64
N_CHUNKS = N_ROWS // CHUNK
PIECE = 128
N_PIECES = N_ROWS // PIECE


def _body(x_ref, meta_ref, d_ref, c1v_ref, out_ref,
          s_ref, r_ref, send_sem, recv_sem):
    my_x = lax.axis_index("x")
    my_y = lax.axis_index("y")
    peer = (my_x, 1 - my_y)

    r_ref[...] = jnp.zeros((N_ROWS, N_COLS), jnp.bfloat16)

    barrier = pltpu.get_barrier_semaphore()
    pl.semaphore_signal(
        barrier, inc=1, device_id=peer,
        device_id_type=pl.DeviceIdType.MESH,
    )

    tot1 = meta_ref[N_ROWS - 1]
    c_send = jnp.where(my_y == 0, tot1, N_ROWS - tot1)
    lo_s = jnp.where(my_y == 0, 0, N_ROWS - c_send)
    hi_s = jnp.where(my_y == 0, c_send, N_ROWS)
    lo_r = jnp.where(my_y == 0, N_ROWS - c_send, 0)
    hi_r = jnp.where(my_y == 0, N_ROWS, c_send)

    d = d_ref[...]
    c1 = c1v_ref[...]
    i_row = lax.broadcasted_iota(jnp.int32, (1, N_ROWS), 1)
    rank = jnp.where(d == 1, c1 - 1, i_row - c1)
    cnt = jnp.where(d == 1, tot1, N_ROWS - tot1)
    pos = jnp.where(my_y == 0, rank, N_ROWS - cnt + rank)
    spos = jnp.where(d != my_y, pos, -1)
    kpos = jnp.where(d == my_y, pos, -1)

    xb = x_ref[...].astype(jnp.bfloat16)

    def compact(off):
        iota_q = lax.broadcasted_iota(jnp.int32, (PIECE, N_ROWS), 0) + off
        p = (spos == iota_q).astype(jnp.bfloat16)
        s_ref[pl.ds(off, PIECE)] = jnp.dot(
            p, xb, preferred_element_type=jnp.float32
        ).astype(jnp.bfloat16)

    def chunk_rdma(j):
        return pltpu.make_async_remote_copy(
            src_ref=s_ref.at[pl.ds(j * CHUNK, CHUNK)],
            dst_ref=r_ref.at[pl.ds(j * CHUNK, CHUNK)],
            send_sem=send_sem,
            recv_sem=recv_sem,
            device_id=peer,
            device_id_type=pl.DeviceIdType.MESH,
        )

    def start_chunks(off):
        for j in range(N_CHUNKS):
            in_q = jnp.logical_and(j * CHUNK >= off, j * CHUNK < off + PIECE)
            in_blk = jnp.logical_and(j * CHUNK < hi_s, (j + 1) * CHUNK > lo_s)
            @pl.when(jnp.logical_and(in_q, in_blk))
            def _(j=j):
                chunk_rdma(j).start()

    offs = [
        jnp.where(my_y == 0, k * PIECE, (N_PIECES - 1 - k) * PIECE)
        for k in range(N_PIECES)
    ]
    compact(offs[0])
    pl.semaphore_wait(barrier, 1)
    start_chunks(offs[0])
    for k in range(1, N_PIECES):
        compact(offs[k])
        start_chunks(offs[k])

    iota = lax.broadcasted_iota(jnp.int32, (N_ROWS, N_ROWS), 0)
    p_keep = (kpos == iota).astype(jnp.bfloat16)
    keep = jnp.dot(
        p_keep, xb, preferred_element_type=jnp.float32
    ).astype(jnp.bfloat16)

    for j in range(N_CHUNKS):
        @pl.when(jnp.logical_and(j * CHUNK < hi_r, (j + 1) * CHUNK > lo_r))
        def _():
            chunk_rdma(0).wait_recv()

    out_ref[...] = keep + r_ref[...]

    for j in range(N_CHUNKS):
        @pl.when(jnp.logical_and(j * CHUNK < hi_s, (j + 1) * CHUNK > lo_s))
        def _():
            chunk_rdma(0).wait_send()


def kernel(x, dest):
    dest = dest.astype(jnp.int32)
    c1 = jnp.cumsum((dest == 1).astype(jnp.int32))
    return pl.pallas_call(
        _body,
        out_shape=jax.ShapeDtypeStruct((N_ROWS, N_COLS), jnp.bfloat16),
        in_specs=[
            pl.BlockSpec(memory_space=pltpu.VMEM),
            pl.BlockSpec(memory_space=pltpu.SMEM),
            pl.BlockSpec(memory_space=pltpu.VMEM),
            pl.BlockSpec(memory_space=pltpu.VMEM),
        ],
        out_specs=pl.BlockSpec(memory_space=pltpu.VMEM),
        scratch_shapes=[
            pltpu.VMEM((N_ROWS, N_COLS), jnp.bfloat16),
            pltpu.VMEM((N_ROWS, N_COLS), jnp.bfloat16),
            pltpu.SemaphoreType.DMA,
            pltpu.SemaphoreType.DMA,
        ],
        compiler_params=pltpu.CompilerParams(collective_id=0),
    )(x, c1, dest.reshape(1, N_ROWS), c1.reshape(1, N_ROWS))
